# baseline (speedup 1.0000x reference)
import jax
import jax.numpy as jnp
from jax import lax
from jax.experimental import pallas as pl
from jax.experimental.pallas import tpu as pltpu

N = 8192
D = 64
BM = 512

_DN_T = (((1,), (1,)), ((), ()))


def _fused_kernel(xs_ref, adj_ref, x_hbm, w_ref, b_ref, o_ref,
                  x_f32, x_bf16, sem):
    i = pl.program_id(0)

    @pl.when(i == 0)
    def _stage_x():
        cp = pltpu.make_async_copy(x_hbm, x_f32, sem)
        cp.start()
        cp.wait()
        x_bf16[...] = x_f32[...].astype(jnp.bfloat16)

    neigh = jnp.dot(
        adj_ref[...].astype(jnp.bfloat16),
        x_bf16[...],
        preferred_element_type=jnp.float32,
    )
    acc = lax.dot_general(xs_ref[...], w_ref[:, :D], _DN_T,
                          preferred_element_type=jnp.float32)
    acc = acc + lax.dot_general(neigh, w_ref[:, D:], _DN_T,
                                preferred_element_type=jnp.float32)
    o_ref[...] = jnp.maximum(acc + b_ref[...], 0.0)


@jax.jit
def kernel(x, adj_matrix, W, b):
    b2 = b.reshape(1, D)
    out = pl.pallas_call(
        _fused_kernel,
        grid=(N // BM,),
        in_specs=[
            pl.BlockSpec((BM, D), lambda i: (i, 0)),
            pl.BlockSpec((BM, N), lambda i: (i, 0)),
            pl.BlockSpec(memory_space=pltpu.HBM),
            pl.BlockSpec((D, 2 * D), lambda i: (0, 0)),
            pl.BlockSpec((1, D), lambda i: (0, 0)),
        ],
        out_specs=pl.BlockSpec((BM, D), lambda i: (i, 0)),
        out_shape=jax.ShapeDtypeStruct((N, D), jnp.float32),
        scratch_shapes=[
            pltpu.VMEM((N, D), jnp.float32),
            pltpu.VMEM((N, D), jnp.bfloat16),
            pltpu.SemaphoreType.DMA,
        ],
        compiler_params=pltpu.CompilerParams(
            dimension_semantics=(pltpu.ARBITRARY,),
            vmem_limit_bytes=60 * 1024 * 1024,
        ),
    )(x, adj_matrix, x, W, b2)
    return out


# speedup vs baseline: 1.0369x; 1.0369x over previous
import jax
import jax.numpy as jnp
from jax import lax
from jax.experimental import pallas as pl
from jax.experimental.pallas import tpu as pltpu

N = 8192
D = 64
BM = 512

_DN_T = (((1,), (1,)), ((), ()))


def _fused_kernel(xs_ref, adj_ref, x_ref, w_ref, b_ref, o_ref):
    neigh = jnp.dot(
        adj_ref[...].astype(jnp.bfloat16),
        x_ref[...].astype(jnp.bfloat16),
        preferred_element_type=jnp.float32,
    )
    acc = lax.dot_general(xs_ref[...], w_ref[:, :D], _DN_T,
                          preferred_element_type=jnp.float32)
    acc = acc + lax.dot_general(neigh, w_ref[:, D:], _DN_T,
                                preferred_element_type=jnp.float32)
    o_ref[...] = jnp.maximum(acc + b_ref[...], 0.0)


@jax.jit
def kernel(x, adj_matrix, W, b):
    b2 = b.reshape(1, D)
    out = pl.pallas_call(
        _fused_kernel,
        grid=(N // BM,),
        in_specs=[
            pl.BlockSpec((BM, D), lambda i: (i, 0)),
            pl.BlockSpec((BM, N), lambda i: (i, 0)),
            pl.BlockSpec((N, D), lambda i: (0, 0)),
            pl.BlockSpec((D, 2 * D), lambda i: (0, 0)),
            pl.BlockSpec((1, D), lambda i: (0, 0)),
        ],
        out_specs=pl.BlockSpec((BM, D), lambda i: (i, 0)),
        out_shape=jax.ShapeDtypeStruct((N, D), jnp.float32),
        compiler_params=pltpu.CompilerParams(
            dimension_semantics=(pltpu.PARALLEL,),
            vmem_limit_bytes=60 * 1024 * 1024,
        ),
    )(x, adj_matrix, x, W, b2)
    return out
